# baseline (speedup 1.0000x reference)
import functools

import jax
import jax.numpy as jnp
from jax import lax
from jax.experimental import pallas as pl
from jax.experimental.pallas import tpu as pltpu
from jax.experimental.pallas import tpu_sc as plsc

VOCAB = 1000000
DIM = 16
BATCH = 16384

_PACK = 128 // DIM
_info = plsc.get_sparse_core_info()
_NC = _info.num_cores
_NS = _info.num_subcores
_NW = _NC * _NS
_B_PER_W = BATCH // _NW
_G = _B_PER_W // 16

_mesh = plsc.VectorSubcoreMesh(core_axis_name="c", subcore_axis_name="s")


@functools.partial(
    pl.kernel,
    mesh=_mesh,
    out_type=jax.ShapeDtypeStruct((BATCH // _PACK, 128), jnp.float32),
    compiler_params=pltpu.CompilerParams(needs_layout_passes=False),
    scratch_types=[
        pltpu.VMEM((_B_PER_W,), jnp.int32),
        pltpu.VMEM((_B_PER_W,), jnp.int32),
        pltpu.VMEM((_B_PER_W, 128), jnp.float32),
        pltpu.VMEM((_B_PER_W // _PACK, 128), jnp.float32),
        pltpu.SemaphoreType.DMA,
    ],
)
def _gather(table_hbm, labels_hbm, out_hbm, idx_v, prow_v, rows_v, out_v, sem):
    wid = lax.axis_index("s") * _NC + lax.axis_index("c")
    base = wid * _B_PER_W
    pltpu.sync_copy(labels_hbm.at[pl.ds(base, _B_PER_W)], idx_v)

    def splat(c):
        return jnp.full((16,), c, jnp.int32)

    for g in range(_G):
        v = idx_v[pl.ds(g * 16, 16)]
        prow_v[pl.ds(g * 16, 16)] = v >> splat(3)

    pltpu.async_copy(table_hbm.at[prow_v], rows_v, sem).wait()

    lanes = lax.broadcasted_iota(jnp.int32, (16,), 0)
    lane_div = lanes >> splat(3)
    lane_col = (lanes & splat(7)) << splat(4)
    for g in range(_G):
        v = idx_v[pl.ds(g * 16, 16)]
        off = (v & splat(7)) << splat(4)
        src_row = splat(g * 16) + lanes
        dst_row = splat(2 * g) + lane_div
        for j in range(DIM):
            vals = plsc.load_gather(rows_v, [src_row, off + splat(j)])
            plsc.store_scatter(out_v, [dst_row, lane_col + splat(j)], vals)

    pltpu.sync_copy(out_v, out_hbm.at[pl.ds(wid * (_B_PER_W // _PACK),
                                            _B_PER_W // _PACK)])


def _iota_body(o_ref):
    o_ref[...] = lax.broadcasted_iota(jnp.int32, o_ref.shape, 0)


_iota_call = pl.pallas_call(
    _iota_body,
    out_shape=jax.ShapeDtypeStruct((VOCAB,), jnp.int32),
)


def kernel(table, stock, labels):
    del stock
    packed = _gather(table.reshape(VOCAB // _PACK, 128), labels)
    data = packed.reshape(BATCH, DIM)
    stock_keys = _iota_call()
    return (data, stock_keys)


# speedup vs baseline: 6.0716x; 6.0716x over previous
import functools

import jax
import jax.numpy as jnp
from jax import lax
from jax.experimental import pallas as pl
from jax.experimental.pallas import tpu as pltpu
from jax.experimental.pallas import tpu_sc as plsc

VOCAB = 1000000
DIM = 16
BATCH = 16384

_info = plsc.get_sparse_core_info()
_NC = _info.num_cores
_NS = _info.num_subcores
_NW = _NC * _NS
_B_PER_W = BATCH // _NW
_GRP = 16
_NG = _B_PER_W // _GRP
_NBANK = 2
_SLOTS = _NBANK * _GRP

_mesh = plsc.VectorSubcoreMesh(core_axis_name="c", subcore_axis_name="s")


@functools.partial(
    pl.kernel,
    mesh=_mesh,
    out_type=jax.ShapeDtypeStruct((DIM, BATCH), jnp.float32),
    compiler_params=pltpu.CompilerParams(needs_layout_passes=False),
    scratch_types=[
        pltpu.VMEM((_B_PER_W,), jnp.int32),
        pltpu.VMEM((DIM, _SLOTS * 128), jnp.float32),
        pltpu.VMEM((DIM, _B_PER_W), jnp.float32),
        pltpu.SemaphoreType.DMA,
        pltpu.SemaphoreType.DMA,
    ],
)
def _gather(tableT_hbm, labels_hbm, outT_hbm, idx_v, stage_v, outT_v,
            sem0, sem1):
    wid = lax.axis_index("s") * _NC + lax.axis_index("c")
    base = wid * _B_PER_W
    pltpu.sync_copy(labels_hbm.at[pl.ds(base, _B_PER_W)], idx_v)

    lanes = lax.broadcasted_iota(jnp.int32, (16,), 0)

    def fire(g, bank, sem):
        vec = idx_v[pl.ds(g * _GRP, _GRP)]
        for k in range(_GRP):
            r = vec[k]
            j = pl.multiple_of((r >> 7) << 7, 128)
            col0 = (bank * _GRP + k) * 128
            pltpu.async_copy(
                tableT_hbm.at[:, pl.ds(j, 128)],
                stage_v.at[:, pl.ds(col0, 128)],
                sem,
            )

    def drain_extract(g, bank, sem):
        vec = idx_v[pl.ds(g * _GRP, _GRP)]
        for k in range(_GRP):
            col0 = (bank * _GRP + k) * 128
            pltpu.make_async_copy(
                tableT_hbm.at[:, pl.ds(0, 128)],
                stage_v.at[:, pl.ds(col0, 128)],
                sem,
            ).wait()
            r = vec[k]
            col = (bank * _GRP + k) * 128 + (r & 127)
            vals = plsc.load_gather(
                stage_v, [lanes, jnp.full((16,), col, jnp.int32)]
            )
            plsc.store_scatter(
                outT_v,
                [lanes, jnp.full((16,), g * _GRP + k, jnp.int32)],
                vals,
            )

    fire(0, 0, sem0)
    fire(1, 1, sem1)

    def body(h, _):
        drain_extract(2 * h, 0, sem0)

        @pl.when(h < _NG // 2 - 1)
        def _():
            fire(2 * h + 2, 0, sem0)

        drain_extract(2 * h + 1, 1, sem1)

        @pl.when(h < _NG // 2 - 1)
        def _():
            fire(2 * h + 3, 1, sem1)

        return 0

    lax.fori_loop(0, _NG // 2, body, 0)

    pltpu.sync_copy(outT_v, outT_hbm.at[:, pl.ds(base, _B_PER_W)])


def _iota_body(o_ref):
    o_ref[...] = lax.broadcasted_iota(jnp.int32, o_ref.shape, 0)


_iota_call = pl.pallas_call(
    _iota_body,
    out_shape=jax.ShapeDtypeStruct((VOCAB,), jnp.int32),
)


def kernel(table, stock, labels):
    del stock
    dataT = _gather(table.T, labels)
    stock_keys = _iota_call()
    return (dataT.T, stock_keys)


# speedup vs baseline: 6.6472x; 1.0948x over previous
import functools

import jax
import jax.numpy as jnp
from jax import lax
from jax.experimental import pallas as pl
from jax.experimental.pallas import tpu as pltpu
from jax.experimental.pallas import tpu_sc as plsc

VOCAB = 1000000
DIM = 16
BATCH = 16384

_info = plsc.get_sparse_core_info()
_NC = _info.num_cores
_NS = _info.num_subcores
_NW = _NC * _NS
_B_PER_W = BATCH // _NW
_GRP = 16
_NG = _B_PER_W // _GRP
_NBANK = 3
_SLOTS = _NBANK * _GRP

_mesh = plsc.VectorSubcoreMesh(core_axis_name="c", subcore_axis_name="s")


@functools.partial(
    pl.kernel,
    mesh=_mesh,
    out_type=jax.ShapeDtypeStruct((DIM, BATCH), jnp.float32),
    compiler_params=pltpu.CompilerParams(needs_layout_passes=False),
    scratch_types=[
        pltpu.VMEM((_B_PER_W,), jnp.int32),
        pltpu.VMEM((DIM, _SLOTS * 128), jnp.float32),
        pltpu.VMEM((DIM, _B_PER_W), jnp.float32),
        pltpu.SemaphoreType.DMA,
        pltpu.SemaphoreType.DMA,
        pltpu.SemaphoreType.DMA,
    ],
)
def _gather(tableT_hbm, labels_hbm, outT_hbm, idx_v, stage_v, outT_v,
            sem0, sem1, sem2):
    wid = lax.axis_index("s") * _NC + lax.axis_index("c")
    base = wid * _B_PER_W
    pltpu.sync_copy(labels_hbm.at[pl.ds(base, _B_PER_W)], idx_v)

    lanes = lax.broadcasted_iota(jnp.int32, (16,), 0)
    sems = (sem0, sem1, sem2)
    slotcols = [(b * _GRP + lanes) * 128 for b in range(_NBANK)]

    def fire(g, bank):
        vec = idx_v[pl.ds(g * _GRP, _GRP)]
        for k in range(_GRP):
            r = vec[k]
            j = pl.multiple_of((r >> 7) << 7, 128)
            col0 = (bank * _GRP + k) * 128
            pltpu.async_copy(
                tableT_hbm.at[:, pl.ds(j, 128)],
                stage_v.at[:, pl.ds(col0, 128)],
                sems[bank],
            )

    def drain_extract(g, bank):
        for k in range(_GRP):
            col0 = (bank * _GRP + k) * 128
            pltpu.make_async_copy(
                tableT_hbm.at[:, pl.ds(0, 128)],
                stage_v.at[:, pl.ds(col0, 128)],
                sems[bank],
            ).wait()
        vec = idx_v[pl.ds(g * _GRP, _GRP)]
        cols = slotcols[bank] + (vec & 127)
        for c in range(DIM):
            vals = plsc.load_gather(
                stage_v, [jnp.full((16,), c, jnp.int32), cols]
            )
            outT_v[c, pl.ds(g * _GRP, _GRP)] = vals

    for b in range(_NBANK):
        fire(b, b)

    def body(h, _):
        for b in range(_NBANK):
            g = _NBANK * h + b
            drain_extract(g, b)

            @pl.when(g + _NBANK < _NG)
            def _():
                fire(g + _NBANK, b)

        return 0

    lax.fori_loop(0, _NG // _NBANK, body, 0)
    for b in range(_NG % _NBANK):
        drain_extract(_NG - _NG % _NBANK + b, b)

    pltpu.sync_copy(outT_v, outT_hbm.at[:, pl.ds(base, _B_PER_W)])


def _iota_body(o_ref):
    o_ref[...] = lax.broadcasted_iota(jnp.int32, o_ref.shape, 0)


_iota_call = pl.pallas_call(
    _iota_body,
    out_shape=jax.ShapeDtypeStruct((VOCAB,), jnp.int32),
)


def kernel(table, stock, labels):
    del stock
    dataT = _gather(table.T, labels)
    stock_keys = _iota_call()
    return (dataT.T, stock_keys)
